# Initial kernel scaffold; baseline (speedup 1.0000x reference)
import jax
import jax.numpy as jnp
from jax.experimental import pallas as pl


def kernel(x, edge_index, W1, b1, W2, b2, fc1_W, fc1_b, fc2_W, fc2_b):
    raise NotImplementedError("write your pallas kernel here")


# speedup vs baseline: 10.8480x; 10.8480x over previous
import functools

import jax
import jax.numpy as jnp
from jax import lax
from jax.experimental import pallas as pl
from jax.experimental.pallas import tpu as pltpu
from jax.experimental.pallas import tpu_sc as plsc

N = 10000
D = 128
E = 320000
NC = 2
NS = 16
NW = NC * NS
CHUNK = 128
K = -(-E // (NW * CHUNK))
E_PAD = NW * K * CHUNK
ACC_ROWS = 10240
RPT = ACC_ROWS // NS
DUMMY = N
DEG_W = 128

NUM_GRAPHS = 100
FC_IN = 12800
FC_HIDDEN = 256
OUT_DIM = 64

_mesh = plsc.VectorSubcoreMesh(core_axis_name="c", subcore_axis_name="s")


@functools.partial(
    pl.kernel,
    out_type=jax.ShapeDtypeStruct((NC, ACC_ROWS, DEG_W), jnp.float32),
    mesh=_mesh,
    scratch_types=[
        pltpu.VMEM((K, CHUNK), jnp.int32),
        pltpu.VMEM((CHUNK, DEG_W), jnp.float32),
        pltpu.VMEM_SHARED((ACC_ROWS, DEG_W), jnp.float32),
    ],
)
def _sc_degree(dst_hbm, ones_hbm, zeros_hbm, out_hbm, dst_v, ones_v, acc):
    cid = lax.axis_index("c")
    sid = lax.axis_index("s")
    wid = cid * NS + sid

    pltpu.sync_copy(zeros_hbm, acc.at[pl.ds(sid * RPT, RPT)])
    pltpu.sync_copy(dst_hbm.at[wid], dst_v)
    pltpu.sync_copy(ones_hbm, ones_v)
    plsc.subcore_barrier()

    def body(j, _):
        pltpu.sync_copy(ones_v, acc.at[dst_v.at[j]], add=True)
        return ()

    lax.fori_loop(0, K, body, (), unroll=False)
    plsc.subcore_barrier()
    pltpu.sync_copy(acc.at[pl.ds(sid * RPT, RPT)],
                    out_hbm.at[cid].at[pl.ds(sid * RPT, RPT)])


@functools.partial(
    pl.kernel,
    out_type=jax.ShapeDtypeStruct((NC, ACC_ROWS, D), jnp.float32),
    mesh=_mesh,
    scratch_types=[
        pltpu.VMEM((K, CHUNK), jnp.int32),
        pltpu.VMEM((K, CHUNK), jnp.int32),
        pltpu.VMEM((CHUNK, D), jnp.float32),
        pltpu.VMEM_SHARED((ACC_ROWS, D), jnp.float32),
        pltpu.SemaphoreType.DMA,
    ],
)
def _sc_scatter(y_hbm, src_hbm, dst_hbm, zeros_hbm, out_hbm,
                src_v, dst_v, rows_v, acc, sem):
    cid = lax.axis_index("c")
    sid = lax.axis_index("s")
    wid = cid * NS + sid

    pltpu.sync_copy(zeros_hbm, acc.at[pl.ds(sid * RPT, RPT)])
    pltpu.sync_copy(src_hbm.at[wid], src_v)
    pltpu.sync_copy(dst_hbm.at[wid], dst_v)
    plsc.subcore_barrier()

    def body(j, _):
        pltpu.async_copy(y_hbm.at[src_v.at[j]], rows_v, sem).wait()
        pltpu.sync_copy(rows_v, acc.at[dst_v.at[j]], add=True)
        return ()

    lax.fori_loop(0, K, body, (), unroll=False)
    plsc.subcore_barrier()
    pltpu.sync_copy(acc.at[pl.ds(sid * RPT, RPT)],
                    out_hbm.at[cid].at[pl.ds(sid * RPT, RPT)])


_R = 1000


def _dis_block(degp):
    deg = degp[0, :, 0] + degp[1, :, 0] + 1.0
    return lax.rsqrt(deg)[:, None]


def _tc_first(x_ref, w1_ref, degp_ref, xw_ref, y_ref):
    dis = _dis_block(degp_ref)
    xw = jnp.dot(x_ref[...], w1_ref[...], preferred_element_type=jnp.float32)
    xw_ref[...] = xw
    y_ref[...] = xw * dis


def _tc_mid(zp_ref, degp_ref, xw1_ref, b1_ref, w2_ref, xw2_ref, y2_ref):
    dis = _dis_block(degp_ref)
    z = zp_ref[0] + zp_ref[1]
    h1 = jnp.maximum(z * dis + xw1_ref[...] * (dis * dis) + b1_ref[...], 0.0)
    xw2 = jnp.dot(h1, w2_ref[...], preferred_element_type=jnp.float32)
    xw2_ref[...] = xw2
    y2_ref[...] = xw2 * dis


def _tc_last(zp_ref, degp_ref, xw2_ref, b2_ref, h2_ref):
    dis = _dis_block(degp_ref)
    z = zp_ref[0] + zp_ref[1]
    h2_ref[...] = jnp.maximum(
        z * dis + xw2_ref[...] * (dis * dis) + b2_ref[...], 0.0)


def _tc_fc(h_ref, w1_ref, b1_ref, w2_ref, b2_ref, o_ref):
    h = jnp.maximum(
        jnp.dot(h_ref[...], w1_ref[...], preferred_element_type=jnp.float32)
        + b1_ref[...], 0.0)
    o_ref[...] = (jnp.dot(h, w2_ref[...], preferred_element_type=jnp.float32)
                  + b2_ref[...])


_rows = pl.BlockSpec((_R, D), lambda i: (i, 0))
_full_w = pl.BlockSpec((D, D), lambda i: (0, 0))
_degs = pl.BlockSpec((NC, _R, DEG_W), lambda i: (0, i, 0))
_parts = pl.BlockSpec((NC, _R, D), lambda i: (0, i, 0))
_bias = pl.BlockSpec((1, D), lambda i: (0, 0))
_f32 = jnp.float32

_first_call = pl.pallas_call(
    _tc_first, grid=(N // _R,),
    in_specs=[_rows, _full_w, _degs],
    out_specs=[_rows, _rows],
    out_shape=[jax.ShapeDtypeStruct((N, D), _f32)] * 2,
)

_mid_call = pl.pallas_call(
    _tc_mid, grid=(N // _R,),
    in_specs=[_parts, _degs, _rows, _bias, _full_w],
    out_specs=[_rows, _rows],
    out_shape=[jax.ShapeDtypeStruct((N, D), _f32)] * 2,
)

_last_call = pl.pallas_call(
    _tc_last, grid=(N // _R,),
    in_specs=[_parts, _degs, _rows, _bias],
    out_specs=_rows,
    out_shape=jax.ShapeDtypeStruct((N, D), _f32),
)

_fc_call = pl.pallas_call(
    _tc_fc,
    out_shape=jax.ShapeDtypeStruct((NUM_GRAPHS, OUT_DIM), _f32),
)


def kernel(x, edge_index, W1, b1, W2, b2, fc1_W, fc1_b, fc2_W, fc2_b):
    src = edge_index[0]
    dst = edge_index[1]
    pad = E_PAD - E
    src_r = jnp.concatenate(
        [src, jnp.zeros((pad,), jnp.int32)]).reshape(NW, K, CHUNK)
    dst_r = jnp.concatenate(
        [dst, jnp.full((pad,), DUMMY, jnp.int32)]).reshape(NW, K, CHUNK)
    ones_blk = jnp.ones((CHUNK, DEG_W), jnp.float32)
    zeros_deg = jnp.zeros((RPT, DEG_W), jnp.float32)
    zeros_row = jnp.zeros((RPT, D), jnp.float32)

    degp = _sc_degree(dst_r, ones_blk, zeros_deg)
    xw1, y1 = _first_call(x, W1, degp)
    z1p = _sc_scatter(y1, src_r, dst_r, zeros_row)
    xw2, y2 = _mid_call(z1p, degp, xw1, b1.reshape(1, D), W2)
    z2p = _sc_scatter(y2, src_r, dst_r, zeros_row)
    h2 = _last_call(z2p, degp, xw2, b2.reshape(1, D))
    return _fc_call(h2.reshape(NUM_GRAPHS, FC_IN), fc1_W,
                    fc1_b.reshape(1, FC_HIDDEN), fc2_W,
                    fc2_b.reshape(1, OUT_DIM))
